# baseline (speedup 1.0000x reference)
import functools

import jax
import jax.numpy as jnp
from jax import lax
from jax.experimental import pallas as pl
from jax.experimental.pallas import tpu as pltpu
from jax.experimental.pallas import tpu_sc as plsc

_BATCH = 4096
_HIDDEN = 1024


@functools.cache
def _build(batch: int, hidden: int, n_rows: int, dtype):
    info = plsc.get_sparse_core_info()
    nc, ns = info.num_cores, info.num_subcores
    nw = nc * ns
    assert batch % nw == 0
    b_per_w = batch // nw
    chunk = 32
    n_chunks = b_per_w // chunk
    mesh = plsc.VectorSubcoreMesh(core_axis_name="c", subcore_axis_name="s")

    @functools.partial(
        pl.kernel,
        mesh=mesh,
        out_type=jax.ShapeDtypeStruct((batch, hidden), dtype),
        scratch_types=[
            pltpu.VMEM((b_per_w,), jnp.int32),
            pltpu.VMEM((chunk, hidden), dtype),
            pltpu.VMEM((chunk, hidden), dtype),
            pltpu.SemaphoreType.DMA,
            pltpu.SemaphoreType.DMA,
            pltpu.SemaphoreType.DMA,
            pltpu.SemaphoreType.DMA,
        ],
    )
    def emb(table_hbm, idx_hbm, out_hbm, idx_v, buf0, buf1, gs0, gs1, ss0, ss1):
        wid = lax.axis_index("s") * nc + lax.axis_index("c")
        base = wid * b_per_w
        pltpu.sync_copy(idx_hbm.at[pl.ds(base, b_per_w)], idx_v)

        bufs = (buf0, buf1)
        gsems = (gs0, gs1)
        ssems = (ss0, ss1)

        def gather(c, b):
            return pltpu.async_copy(
                table_hbm.at[idx_v.at[pl.ds(c * chunk, chunk)]], bufs[b], gsems[b]
            )

        def store(c, b):
            return pltpu.async_copy(
                bufs[b], out_hbm.at[pl.ds(base + c * chunk, chunk)], ssems[b]
            )

        gh = [None, None]
        sh = [None, None]
        for c in range(n_chunks):
            b = c & 1
            if sh[b] is not None:
                sh[b].wait()
                sh[b] = None
            gh[b] = gather(c, b)
            if c >= 1:
                pb = (c - 1) & 1
                gh[pb].wait()
                sh[pb] = store(c - 1, pb)
        lb = (n_chunks - 1) & 1
        gh[lb].wait()
        sh[lb] = store(n_chunks - 1, lb)
        for b in range(2):
            if sh[b] is not None:
                sh[b].wait()

    return emb


def kernel(labels, embedding_table):
    n_rows, hidden = embedding_table.shape
    emb = _build(labels.shape[0], hidden, n_rows, embedding_table.dtype)
    return emb(embedding_table, labels)


# speedup vs baseline: 1.0124x; 1.0124x over previous
import functools

import jax
import jax.numpy as jnp
from jax import lax
from jax.experimental import pallas as pl
from jax.experimental.pallas import tpu as pltpu
from jax.experimental.pallas import tpu_sc as plsc

_BATCH = 4096
_HIDDEN = 1024


@functools.cache
def _build(batch: int, hidden: int, n_rows: int, dtype):
    info = plsc.get_sparse_core_info()
    nc, ns = info.num_cores, info.num_subcores
    nw = nc * ns
    assert batch % nw == 0
    b_per_w = batch // nw
    chunk = 32
    n_chunks = b_per_w // chunk
    mesh = plsc.VectorSubcoreMesh(core_axis_name="c", subcore_axis_name="s")

    @functools.partial(
        pl.kernel,
        mesh=mesh,
        out_type=jax.ShapeDtypeStruct((batch, hidden), dtype),
        scratch_types=[
            pltpu.VMEM((b_per_w,), jnp.int32),
            pltpu.VMEM((chunk, hidden), dtype),
            pltpu.VMEM((chunk, hidden), dtype),
            pltpu.VMEM((chunk, hidden), dtype),
            pltpu.SemaphoreType.DMA,
            pltpu.SemaphoreType.DMA,
            pltpu.SemaphoreType.DMA,
            pltpu.SemaphoreType.DMA,
            pltpu.SemaphoreType.DMA,
            pltpu.SemaphoreType.DMA,
        ],
    )
    def emb(
        table_hbm, idx_hbm, out_hbm,
        idx_v, buf0, buf1, buf2, gs0, gs1, gs2, ss0, ss1, ss2,
    ):
        wid = lax.axis_index("s") * nc + lax.axis_index("c")
        base = wid * b_per_w
        pltpu.sync_copy(idx_hbm.at[pl.ds(base, b_per_w)], idx_v)

        bufs = (buf0, buf1, buf2)
        gsems = (gs0, gs1, gs2)
        ssems = (ss0, ss1, ss2)
        nbuf = len(bufs)

        def gather(c):
            b = c % nbuf
            return pltpu.async_copy(
                table_hbm.at[idx_v.at[pl.ds(c * chunk, chunk)]], bufs[b], gsems[b]
            )

        def store(c):
            b = c % nbuf
            return pltpu.async_copy(
                bufs[b], out_hbm.at[pl.ds(base + c * chunk, chunk)], ssems[b]
            )

        gh = {}
        sh = {}
        for c in range(min(nbuf, n_chunks)):
            gh[c] = gather(c)
        for c in range(n_chunks):
            gh[c].wait()
            sh[c] = store(c)
            nxt = c + nbuf
            if nxt < n_chunks:
                sh[c].wait()
                sh.pop(c)
                gh[nxt] = gather(nxt)
        for c, h in sh.items():
            h.wait()

    return emb


def kernel(labels, embedding_table):
    n_rows, hidden = embedding_table.shape
    emb = _build(labels.shape[0], hidden, n_rows, embedding_table.dtype)
    return emb(embedding_table, labels)
